# Initial kernel scaffold; baseline (speedup 1.0000x reference)
import jax
import jax.numpy as jnp
from jax.experimental import pallas as pl


def kernel(x1, edge_index1, e1, u1, batch1, x2, edge_index2, e2, u2, batch2, params):
    raise NotImplementedError("write your pallas kernel here")


# speedup vs baseline: 3.1514x; 3.1514x over previous
import functools

import jax
import jax.numpy as jnp
from jax import lax
from jax.experimental import pallas as pl
from jax.experimental.pallas import tpu as pltpu
from jax.experimental.pallas import tpu_sc as plsc

F32 = jnp.float32
NN = 10000
NE = 160000
NP = 10240
EP = 163840
NG = 64
GB = 80
NWK = 32
EPW = EP // NWK
NCH = EPW // 128
NPW = NP // 16

@functools.lru_cache(maxsize=None)
def _sc_mesh():
    return plsc.VectorSubcoreMesh(core_axis_name="c", subcore_axis_name="s")


def _wid():
    return lax.axis_index("s") * 2 + lax.axis_index("c")


def _stats_body(dest_hbm, gid_hbm, zeros_hbm, ones_hbm,
                hd_hbm, hg_hbm,
                dest_v, gid_v, ones_v, hd_acc, hg_acc):
    cid = lax.axis_index("c")
    sid = lax.axis_index("s")
    wid = sid * 2 + cid
    pltpu.sync_copy(zeros_hbm, hd_acc.at[pl.ds(sid * NPW, NPW)])
    @pl.when(sid == 0)
    def _():
        pltpu.sync_copy(zeros_hbm.at[pl.ds(0, GB)], hg_acc)
    pltpu.sync_copy(ones_hbm, ones_v)
    plsc.subcore_barrier()

    def chunk(g, carry):
        base = wid * EPW + g * 128
        pltpu.sync_copy(dest_hbm.at[pl.ds(base, 128)], dest_v)
        pltpu.sync_copy(gid_hbm.at[pl.ds(base, 128)], gid_v)
        pltpu.sync_copy(ones_v, hd_acc.at[dest_v], add=True)
        pltpu.sync_copy(ones_v, hg_acc.at[gid_v], add=True)
        return carry

    lax.fori_loop(0, NCH, chunk, 0)
    plsc.subcore_barrier()
    pltpu.sync_copy(hd_acc.at[pl.ds(sid * NPW, NPW)],
                    hd_hbm.at[cid, pl.ds(sid * NPW, NPW)])
    @pl.when(sid == 0)
    def _():
        pltpu.sync_copy(hg_acc, hg_hbm.at[cid])


@functools.lru_cache(maxsize=None)
def _stats_kernel():
    return pl.kernel(
        _stats_body,
        out_type=(jax.ShapeDtypeStruct((2, NP, 128), F32),
                  jax.ShapeDtypeStruct((2, GB, 128), F32)),
        scratch_types=[
            pltpu.VMEM((128,), jnp.int32),
            pltpu.VMEM((128,), jnp.int32),
            pltpu.VMEM((128, 128), F32),
            pltpu.VMEM_SHARED((NP, 128), F32),
            pltpu.VMEM_SHARED((GB, 128), F32),
        ],
        mesh=_sc_mesh(),
    )


def _stats_call(*args):
    return _stats_kernel()(*args)


def _off_tc(b_ref, off_ref):
    i = pl.program_id(0)
    le = (b_ref[...] <= lax.broadcasted_iota(jnp.int32, (1, NG), 1)
          ).astype(jnp.int32)
    c = jnp.sum(le, axis=0)[None, :]

    @pl.when(i == 0)
    def _():
        off_ref[...] = jnp.zeros_like(off_ref)

    off_ref[...] += c


def _off_call(b2d):
    return pl.pallas_call(
        _off_tc,
        grid=(NP // RN,),
        in_specs=[pl.BlockSpec((RN, 1), lambda i: (i, 0))],
        out_specs=pl.BlockSpec((1, NG), lambda i: (0, 0)),
        out_shape=jax.ShapeDtypeStruct((1, NG), jnp.int32),
    )(b2d)


def _gid_tc(src_ref, off_ref, gid_ref):
    ge = (src_ref[...] >= off_ref[...]).astype(jnp.int32)
    gid_ref[...] = jnp.sum(ge, axis=1)[:, None]


def _gid_call(src2d, off):
    return pl.pallas_call(
        _gid_tc,
        grid=(EP // RE,),
        in_specs=[pl.BlockSpec((RE, 1), lambda i: (i, 0)),
                  pl.BlockSpec((1, NG), lambda i: (0, 0))],
        out_specs=pl.BlockSpec((RE, 1), lambda i: (i, 0)),
        out_shape=jax.ShapeDtypeStruct((EP, 1), jnp.int32),
    )(src2d, off)


def _gather_body(p_hbm, q_hbm, dest_hbm, src_hbm, gp_hbm, gq_hbm,
                 idxd_v, idxs_v, rowp_v, rowq_v, semp, semq):
    wid = _wid()

    def chunk(g, carry):
        base = wid * EPW + g * 128
        pltpu.sync_copy(dest_hbm.at[pl.ds(base, 128)], idxd_v)
        pltpu.sync_copy(src_hbm.at[pl.ds(base, 128)], idxs_v)
        cp = pltpu.async_copy(p_hbm.at[idxd_v], rowp_v, semp)
        cq = pltpu.async_copy(q_hbm.at[idxs_v], rowq_v, semq)
        cp.wait()
        cq.wait()
        pltpu.sync_copy(rowp_v, gp_hbm.at[pl.ds(base, 128)])
        pltpu.sync_copy(rowq_v, gq_hbm.at[pl.ds(base, 128)])
        return carry

    lax.fori_loop(0, NCH, chunk, 0)


@functools.lru_cache(maxsize=None)
def _gather_kernel():
    return pl.kernel(
        _gather_body,
        out_type=(jax.ShapeDtypeStruct((EP, 128), F32),
                  jax.ShapeDtypeStruct((EP, 128), F32)),
        scratch_types=[
            pltpu.VMEM((128,), jnp.int32),
            pltpu.VMEM((128,), jnp.int32),
            pltpu.VMEM((128, 128), F32),
            pltpu.VMEM((128, 128), F32),
            pltpu.SemaphoreType.DMA,
            pltpu.SemaphoreType.DMA,
        ],
        mesh=_sc_mesh(),
    )


def _gather_call(*args):
    return _gather_kernel()(*args)


def _scatter_body(rows_hbm, dest_hbm, gid_hbm, zeros_hbm,
                  nout_hbm, gout_hbm,
                  idxd_v, idxg_v, rows_v, node_acc, graph_acc):
    cid = lax.axis_index("c")
    sid = lax.axis_index("s")
    wid = sid * 2 + cid
    pltpu.sync_copy(zeros_hbm, node_acc.at[pl.ds(sid * NPW, NPW)])
    @pl.when(sid == 0)
    def _():
        pltpu.sync_copy(zeros_hbm.at[pl.ds(0, GB)], graph_acc)
    plsc.subcore_barrier()

    def chunk(g, carry):
        base = wid * EPW + g * 128
        pltpu.sync_copy(rows_hbm.at[pl.ds(base, 128)], rows_v)
        pltpu.sync_copy(dest_hbm.at[pl.ds(base, 128)], idxd_v)
        pltpu.sync_copy(gid_hbm.at[pl.ds(base, 128)], idxg_v)
        pltpu.sync_copy(rows_v, node_acc.at[idxd_v], add=True)
        pltpu.sync_copy(rows_v, graph_acc.at[idxg_v], add=True)
        return carry

    lax.fori_loop(0, NCH, chunk, 0)
    plsc.subcore_barrier()
    pltpu.sync_copy(node_acc.at[pl.ds(sid * NPW, NPW)],
                    nout_hbm.at[cid, pl.ds(sid * NPW, NPW)])
    @pl.when(sid == 0)
    def _():
        pltpu.sync_copy(graph_acc, gout_hbm.at[cid])


@functools.lru_cache(maxsize=None)
def _scatter_kernel():
    return pl.kernel(
        _scatter_body,
        out_type=(jax.ShapeDtypeStruct((2, NP, 128), F32),
                  jax.ShapeDtypeStruct((2, GB, 128), F32)),
        scratch_types=[
            pltpu.VMEM((128,), jnp.int32),
            pltpu.VMEM((128,), jnp.int32),
            pltpu.VMEM((128, 128), F32),
            pltpu.VMEM_SHARED((NP, 128), F32),
            pltpu.VMEM_SHARED((GB, 128), F32),
        ],
        mesh=_sc_mesh(),
    )


def _scatter_call(*args):
    return _scatter_kernel()(*args)


RN = 1024


def _prep_tc(x_ref, b_ref, ua_ref, ub_ref, wp_ref, wa_ref, wb_ref, wc_ref,
             p_ref, q_ref):
    x = x_ref[...]
    p0 = jnp.dot(x, wp_ref[...], preferred_element_type=F32)
    ca = jnp.dot(ua_ref[...], wa_ref[...], preferred_element_type=F32)
    cb = (jnp.dot(ua_ref[...], wb_ref[...], preferred_element_type=F32)
          + jnp.dot(ub_ref[...], wc_ref[...], preferred_element_type=F32))
    oh = (b_ref[...] == lax.broadcasted_iota(jnp.int32, (1, NG), 1)).astype(F32)
    p = p0 + jnp.dot(oh, ca, preferred_element_type=F32)
    p_ref[...] = p
    q_ref[...] = jnp.dot(oh, cb, preferred_element_type=F32) - p


def _prep_call(x, b2d, ua, ub, wp, wa, wb, wc):
    small = pl.BlockSpec((NG, 128), lambda i: (0, 0))
    wspec = pl.BlockSpec((128, 128), lambda i: (0, 0))
    return pl.pallas_call(
        _prep_tc,
        grid=(NP // RN,),
        in_specs=[pl.BlockSpec((RN, 128), lambda i: (i, 0)),
                  pl.BlockSpec((RN, 1), lambda i: (i, 0)),
                  small, small, wspec, wspec, wspec, wspec],
        out_specs=[pl.BlockSpec((RN, 128), lambda i: (i, 0))] * 2,
        out_shape=[jax.ShapeDtypeStruct((NP, 128), F32)] * 2,
    )(x, b2d, ua, ub, wp, wa, wb, wc)


RE = 1024


def _edge_tc(e_ref, gp_ref, gq_ref, a_ref, b1_ref, w2_ref, b2_ref, out_ref):
    h = jnp.dot(e_ref[...], a_ref[...], preferred_element_type=F32)
    h = jnp.maximum(h + gp_ref[...] + gq_ref[...] + b1_ref[...], 0.0)
    out_ref[...] = jnp.dot(h, w2_ref[...], preferred_element_type=F32) + b2_ref[...]


def _edge_call(e, gp, gq, a, b1, w2, b2):
    big = pl.BlockSpec((RE, 128), lambda i: (i, 0))
    wspec = pl.BlockSpec((128, 128), lambda i: (0, 0))
    bspec = pl.BlockSpec((1, 128), lambda i: (0, 0))
    return pl.pallas_call(
        _edge_tc,
        grid=(EP // RE,),
        in_specs=[big, big, big, wspec, bspec, wspec, bspec],
        out_specs=big,
        out_shape=jax.ShapeDtypeStruct((EP, 128), F32),
    )(e, gp, gq, a, b1, w2, b2)


def _node_tc(np0_ref, np1_ref, hd0_ref, hd1_ref, x_ref, b_ref, ua_ref, ub_ref,
             wagg_ref, wx_ref, wua_ref, wub_ref, b1_ref, w2_ref, b2_ref,
             xn_ref, xg_ref, nb_ref):
    i = pl.program_id(0)
    cnt = jnp.maximum(hd0_ref[...] + hd1_ref[...], 1.0)
    agg = (np0_ref[...] + np1_ref[...]) / cnt
    ru = (jnp.dot(ua_ref[...], wua_ref[...], preferred_element_type=F32)
          + jnp.dot(ub_ref[...], wub_ref[...], preferred_element_type=F32))
    oh = (b_ref[...] == lax.broadcasted_iota(jnp.int32, (1, NG), 1)).astype(F32)
    h = jnp.dot(agg, wagg_ref[...], preferred_element_type=F32)
    h = h + jnp.dot(x_ref[...], wx_ref[...], preferred_element_type=F32)
    h = h + jnp.dot(oh, ru, preferred_element_type=F32) + b1_ref[...]
    h = jnp.maximum(h, 0.0)
    xn = jnp.dot(h, w2_ref[...], preferred_element_type=F32) + b2_ref[...]
    xn_ref[...] = xn
    xgc = lax.dot_general(oh, xn, (((0,), (0,)), ((), ())),
                          preferred_element_type=F32)
    nbc = jnp.broadcast_to(jnp.sum(oh, axis=0)[:, None], (NG, 128))

    @pl.when(i == 0)
    def _():
        xg_ref[...] = jnp.zeros_like(xg_ref)
        nb_ref[...] = jnp.zeros_like(nb_ref)

    xg_ref[...] += xgc
    nb_ref[...] += nbc


def _node_call(np0, np1, hd0, hd1, x, b2d, ua, ub, wagg, wx, wua, wub, b1, w2, b2):
    big = pl.BlockSpec((RN, 128), lambda i: (i, 0))
    hspec = pl.BlockSpec((RN, 1), lambda i: (i, 0))
    small = pl.BlockSpec((NG, 128), lambda i: (0, 0))
    wspec = pl.BlockSpec((128, 128), lambda i: (0, 0))
    bspec = pl.BlockSpec((1, 128), lambda i: (0, 0))
    return pl.pallas_call(
        _node_tc,
        grid=(NP // RN,),
        in_specs=[big, big, hspec, hspec, big,
                  pl.BlockSpec((RN, 1), lambda i: (i, 0)),
                  small, small, wspec, wspec, wspec, wspec, bspec, wspec, bspec],
        out_specs=[big, small, small],
        out_shape=[jax.ShapeDtypeStruct((NP, 128), F32),
                   jax.ShapeDtypeStruct((NG, 128), F32),
                   jax.ShapeDtypeStruct((NG, 128), F32)],
    )(np0, np1, hd0, hd1, x, b2d, ua, ub, wagg, wx, wua, wub, b1, w2, b2)


def _glob1_tc(gp0_ref, gp1_ref, hg0_ref, hg1_ref, xg_ref, nb_ref, u_ref,
              w1e_ref, w1x_ref, w1u_ref, b1_ref, w2_ref, b2_ref, out_ref):
    cg = jnp.maximum(hg0_ref[...] + hg1_ref[...], 1.0)
    e_g = (gp0_ref[...] + gp1_ref[...]) / cg
    x_g = xg_ref[...] / jnp.maximum(nb_ref[...][:, :1], 1.0)
    h = (jnp.dot(e_g, w1e_ref[...], preferred_element_type=F32)
         + jnp.dot(x_g, w1x_ref[...], preferred_element_type=F32)
         + jnp.dot(u_ref[...], w1u_ref[...], preferred_element_type=F32)
         + b1_ref[...])
    h = jnp.maximum(h, 0.0)
    out_ref[...] = jnp.dot(h, w2_ref[...], preferred_element_type=F32) + b2_ref[...]


def _glob1_call(gp0, gp1, hg0, hg1, xg, nb, u, w1e, w1x, w1u, b1, w2, b2):
    return pl.pallas_call(
        _glob1_tc,
        out_shape=jax.ShapeDtypeStruct((NG, 128), F32),
    )(gp0, gp1, hg0, hg1, xg, nb, u, w1e, w1x, w1u, b1, w2, b2)


def _glob2_tc(gp0_ref, gp1_ref, hg0_ref, hg1_ref, xg_ref, nb_ref,
              u2_ref, u1_ref,
              w1e_ref, w1x_ref, w1a_ref, w1b_ref, b1_ref, w2_ref, b2_ref,
              wr_ref, br_ref, wo1_ref, bo1_ref, wo2_ref, bo2_ref,
              unew_ref, u1r_ref, out_ref):
    cg = jnp.maximum(hg0_ref[...] + hg1_ref[...], 1.0)
    e_g = (gp0_ref[...] + gp1_ref[...]) / cg
    x_g = xg_ref[...] / jnp.maximum(nb_ref[...][:, :1], 1.0)
    h = (jnp.dot(e_g, w1e_ref[...], preferred_element_type=F32)
         + jnp.dot(x_g, w1x_ref[...], preferred_element_type=F32)
         + jnp.dot(u2_ref[...], w1a_ref[...], preferred_element_type=F32)
         + jnp.dot(u1_ref[...], w1b_ref[...], preferred_element_type=F32)
         + b1_ref[...])
    h = jnp.maximum(h, 0.0)
    unew = jnp.dot(h, w2_ref[...], preferred_element_type=F32) + b2_ref[...]
    unew_ref[...] = unew
    u1r_ref[...] = (jnp.dot(jnp.maximum(u1_ref[...], 0.0), wr_ref[...],
                            preferred_element_type=F32) + br_ref[...])
    ho = jnp.maximum(jnp.dot(unew, wo1_ref[...], preferred_element_type=F32)
                     + bo1_ref[...], 0.0)
    out_ref[...] = jnp.dot(ho, wo2_ref[...], preferred_element_type=F32) + bo2_ref[...]


def _glob2_call(gp0, gp1, hg0, hg1, xg, nb, u2, u1,
                w1e, w1x, w1a, w1b, b1, w2, b2, wr, br, wo1, bo1, wo2, bo2):
    return pl.pallas_call(
        _glob2_tc,
        out_shape=[jax.ShapeDtypeStruct((NG, 128), F32)] * 3,
    )(gp0, gp1, hg0, hg1, xg, nb, u2, u1,
      w1e, w1x, w1a, w1b, b1, w2, b2, wr, br, wo1, bo1, wo2, bo2)


def kernel(x1, edge_index1, e1, u1, batch1, x2, edge_index2, e2, u2, batch2,
           params):
    i32 = jnp.int32

    def pad_idx(a, val):
        a = a.astype(i32)
        return jnp.concatenate([a, jnp.full((EP - NE,), val, i32)])

    src1 = pad_idx(edge_index1[0], NN)
    dest1 = pad_idx(edge_index1[1], NN)
    src2 = pad_idx(edge_index2[0], NN)
    dest2 = pad_idx(edge_index2[1], NN)
    bp1 = jnp.concatenate([batch1.astype(i32), jnp.full((NP - NN,), NG, i32)])
    bp2 = jnp.concatenate([batch2.astype(i32), jnp.full((NP - NN,), NG, i32)])
    b2d1 = bp1[:, None]
    b2d2 = bp2[:, None]
    xp1 = jnp.concatenate([x1, jnp.zeros((NP - NN, 128), F32)], axis=0)
    xp2 = jnp.concatenate([x2, jnp.zeros((NP - NN, 128), F32)], axis=0)
    ep1 = jnp.concatenate([e1, jnp.zeros((EP - NE, 128), F32)], axis=0)
    ep2 = jnp.concatenate([e2, jnp.zeros((EP - NE, 128), F32)], axis=0)

    z_n128 = jnp.zeros((NPW, 128), F32)
    o_128 = jnp.ones((128, 128), F32)
    zw = jnp.zeros((128, 128), F32)

    def mlp(ps):
        (w1, b1), (w2, b2) = ps
        return w1, b1[None, :], w2, b2[None, :]

    qeW1, qeb1, qeW2, qeb2 = mlp(params["qe"])
    qxW1, qxb1, qxW2, qxb2 = mlp(params["qx"])
    quW1, qub1, quW2, qub2 = mlp(params["qu"])
    weW1, web1, weW2, web2 = mlp(params["we"])
    wxW1, wxb1, wxW2, wxb2 = mlp(params["wx"])
    wuW1, wub1, wuW2, wub2 = mlp(params["wu"])
    rW, rb = params["ru"]
    rb = rb[None, :]
    oW1, ob1, oW2, ob2 = mlp(params["out"])
    oW2p = jnp.pad(oW2, ((0, 0), (0, 128 - oW2.shape[1])))
    ob2p = jnp.pad(ob2, ((0, 0), (0, 128 - ob2.shape[1])))

    qAe, qAx, qAu = qeW1[0:128], qeW1[128:256], qeW1[256:384]
    qNa, qNx, qNu = qxW1[0:128], qxW1[128:256], qxW1[256:384]
    qGe, qGx, qGu = quW1[0:128], quW1[128:256], quW1[256:384]
    wA1, wA2 = weW1[0:128], weW1[128:256]
    wA3, wA4 = weW1[256:384], weW1[384:512]
    wA5, wA6 = weW1[512:640], weW1[640:768]
    wA26 = wA2 + wA6
    wNa, wNx = wxW1[0:128], wxW1[128:256]
    wNu1, wNu2, wNu1b = wxW1[256:384], wxW1[384:512], wxW1[512:640]
    wNu13 = wNu1 + wNu1b
    wGe, wGx, wGu2, wGu1 = wuW1[0:128], wuW1[128:256], wuW1[256:384], wuW1[384:512]

    gid1 = _gid_call(src1[:, None], _off_call(b2d1)).reshape(EP)
    gid2 = _gid_call(src2[:, None], _off_call(b2d2)).reshape(EP)
    hd1, hg1 = _stats_call(dest1, gid1, z_n128, o_128)
    hd2, hg2 = _stats_call(dest2, gid2, z_n128, o_128)
    hd1, hg1 = hd1[:, :, :1], hg1[:, :, :1]
    hd2, hg2 = hd2[:, :, :1], hg2[:, :, :1]

    def edge_phase(xp, ep, b2d, destg, srcg, gid, ua, ub, wp, wa, wb, wc,
                   ae, eb1, ew2, eb2):
        p, q = _prep_call(xp, b2d, ua, ub, wp, wa, wb, wc)
        gp, gq = _gather_call(p, q, destg, srcg)
        e_new = _edge_call(ep, gp, gq, ae, eb1, ew2, eb2)
        npart, gpart = _scatter_call(e_new, destg, gid, z_n128)
        return e_new, npart, gpart

    zu = jnp.zeros((NG, 128), F32)
    outs = []
    for _ in range(2):
        ep1, npart, gpart = edge_phase(
            xp1, ep1, b2d1, dest1, src1, gid1,
            u1, zu, qAx, zw, qAu, zw, qAe, qeb1, qeW2, qeb2)
        xp1, xg, nb = _node_call(
            npart[0], npart[1], hd1[0], hd1[1], xp1, b2d1,
            u1, zu, qNa, qNx, qNu, zw, qxb1, qxW2, qxb2)
        u1 = _glob1_call(
            gpart[0, :NG], gpart[1, :NG], hg1[0, :NG], hg1[1, :NG], xg, nb, u1,
            qGe, qGx, qGu, qub1, quW2, qub2)
        ep2, npart, gpart = edge_phase(
            xp2, ep2, b2d2, dest2, src2, gid2,
            u1, u2, wA3, wA4, wA26, wA5, wA1, web1, weW2, web2)
        xp2, xg, nb = _node_call(
            npart[0], npart[1], hd2[0], hd2[1], xp2, b2d2,
            u1, u2, wNa, wNx, wNu13, wNu2, wxb1, wxW2, wxb2)
        u2, u1, out_s = _glob2_call(
            gpart[0, :NG], gpart[1, :NG], hg2[0, :NG], hg2[1, :NG], xg, nb,
            u2, u1, wGe, wGx, wGu2, wGu1, wub1, wuW2, wub2,
            rW, rb, oW1, ob1, oW2p, ob2p)
        outs.append(out_s[:, :2])

    return jnp.stack(outs, axis=0)
